# baseline (speedup 1.0000x reference)
import functools

import jax
import jax.numpy as jnp
from jax import lax
from jax.experimental import pallas as pl
from jax.experimental.pallas import tpu as pltpu
from jax.experimental.pallas import tpu_sc as plsc

B, C, H, W = 2, 128, 60, 80
HW = H * W
NVOX = 60 * 36 * 60
TPAD = HW + 16
NTILES = 32
CPT = (B * C) // NTILES
VC = 3600
NCHUNK = NVOX // VC


def _sc_body(feat, idx, out, table_v, idx_v, out_v,
             sem_idx0, sem_idx1, sem_out0, sem_out1):
    cid = lax.axis_index("c")
    sid = lax.axis_index("s")
    wid = sid * 2 + cid
    b = wid // (NTILES // B)
    cbase = (wid % (NTILES // B)) * CPT
    row0 = b * C + cbase

    sem_idx = (sem_idx0, sem_idx1)
    sem_out = (sem_out0, sem_out1)

    def idx_desc(k, slot):
        return pltpu.make_async_copy(
            idx.at[pl.ds(b * NVOX + k * VC, VC)],
            idx_v.at[pl.ds(slot * VC, VC)],
            sem_idx[slot])

    def out_desc(k, slot, j):
        return pltpu.make_async_copy(
            out_v.at[pl.ds((slot * CPT + j) * VC, VC)],
            out.at[pl.ds((row0 + j) * NVOX + k * VC, VC)],
            sem_out[slot])

    zeros16 = jnp.zeros((16,), jnp.float32)
    for j in range(CPT):
        pltpu.sync_copy(feat.at[pl.ds((row0 + j) * HW, HW)],
                        table_v.at[pl.ds(j * TPAD, HW)])
        table_v[pl.ds(j * TPAD + HW, 16)] = zeros16

    idx_desc(0, 0).start()

    @pl.loop(0, NCHUNK, step=2)
    def chunk_pair(k0):
        for p in range(2):
            k = k0 + p
            knext = jnp.minimum(k + 1, NCHUNK - 1)
            idx_desc(knext, 1 - p).start()
            idx_desc(k, p).wait()
            @pl.when(k0 >= 2)
            def _():
                for j in range(CPT):
                    out_desc(k, p, j).wait()

            base0 = p * CPT * VC

            @plsc.parallel_loop(0, VC, 16, unroll=8)
            def gather_body(base):
                iv = idx_v[pl.ds(p * VC + base, 16)]
                for j in range(CPT):
                    out_v[pl.ds(base0 + j * VC + base, 16)] = (
                        plsc.load_gather(
                            table_v.at[pl.ds(j * TPAD, TPAD)], [iv]))

            for j in range(CPT):
                out_desc(k, p, j).start()

    idx_desc(NCHUNK - 1, 0).wait()
    for p in range(2):
        for j in range(CPT):
            out_desc(NCHUNK - 2 + p, p, j).wait()


_sc_call = pl.kernel(
    _sc_body,
    mesh=plsc.VectorSubcoreMesh(core_axis_name="c", subcore_axis_name="s"),
    compiler_params=pltpu.CompilerParams(needs_layout_passes=False),
    out_type=jax.ShapeDtypeStruct((B * C * NVOX,), jnp.float32),
    scratch_types=[
        pltpu.VMEM((CPT * TPAD,), jnp.float32),
        pltpu.VMEM((2 * VC,), jnp.int32),
        pltpu.VMEM((2 * CPT * VC,), jnp.float32),
        pltpu.SemaphoreType.DMA,
        pltpu.SemaphoreType.DMA,
        pltpu.SemaphoreType.DMA,
        pltpu.SemaphoreType.DMA,
    ],
)


@jax.jit
def kernel(feature2d, depth_mapping_3d):
    feat = feature2d.reshape(B * C * HW)
    out = _sc_call(feat, depth_mapping_3d.reshape(B * NVOX))
    return out.reshape(B, C, 60, 36, 60)


# speedup vs baseline: 3.3611x; 3.3611x over previous
import functools

import jax
import jax.numpy as jnp
from jax import lax
from jax.experimental import pallas as pl
from jax.experimental.pallas import tpu as pltpu
from jax.experimental.pallas import tpu_sc as plsc

B, C, H, W = 2, 128, 60, 80
HW = H * W
NVOX = 60 * 36 * 60
NTILES = 32
ROWS1 = 4808
TROWS = 2 * ROWS1
VCH = 192
NR = 2 * VCH
NCHUNK = NVOX // VCH
NITER = (NCHUNK + NTILES - 1) // NTILES


def _sc_body(feat, idx, out, table_s, zbuf, idxb, cidx, rows_v, gsem):
    cid = lax.axis_index("c")
    sid = lax.axis_index("s")
    wid = sid * 2 + cid

    zeros16 = jnp.zeros((16,), jnp.float32)
    for i in range(8):
        for j in range(8):
            zbuf[i, pl.ds(j * 16, 16)] = zeros16
    bsel = jnp.where(sid < 8, 0, 1)
    src0 = pl.multiple_of(bsel * HW + (sid % 8) * 600, 8)
    dst0 = pl.multiple_of(bsel * ROWS1 + (sid % 8) * 600, 8)
    pltpu.sync_copy(feat.at[pl.ds(src0, 600), :],
                    table_s.at[pl.ds(dst0, 600), :])
    @pl.when(sid == 0)
    def _():
        pltpu.sync_copy(zbuf, table_s.at[pl.ds(HW, 8), :])
        pltpu.sync_copy(zbuf, table_s.at[pl.ds(ROWS1 + HW, 8), :])
    plsc.subcore_barrier()

    iota2 = lax.iota(jnp.int32, 16) * 2

    def round_body(t, carry):
        ck = wid + NTILES * t

        @pl.when(ck < NCHUNK)
        def _():
            v0 = ck * VCH
            pltpu.sync_copy(idx.at[pl.ds(v0, VCH)], idxb.at[pl.ds(0, VCH)])
            pltpu.sync_copy(idx.at[pl.ds(NVOX + v0, VCH)],
                            idxb.at[pl.ds(VCH, VCH)])

            def build_body(g, carry2):
                pos = iota2 + g * 32
                iv0 = idxb[pl.ds(g * 16, 16)]
                iv1 = idxb[pl.ds(VCH + g * 16, 16)] + ROWS1
                plsc.store_scatter(cidx, [pos], iv0)
                plsc.store_scatter(cidx, [pos + 1], iv1)
                return carry2

            lax.fori_loop(0, VCH // 16, build_body, 0)

            descs = [pltpu.async_copy(
                table_s.at[cidx.at[pl.ds(j * 128, 128)]],
                rows_v.at[pl.ds(j * 128, 128), :],
                gsem) for j in range(NR // 128)]
            for d in descs:
                d.wait()
            pltpu.sync_copy(rows_v, out.at[pl.ds(v0 * 2, NR), :])
        return carry

    lax.fori_loop(0, NITER, round_body, 0)


_sc_call = pl.kernel(
    _sc_body,
    mesh=plsc.VectorSubcoreMesh(core_axis_name="c", subcore_axis_name="s"),
    compiler_params=pltpu.CompilerParams(needs_layout_passes=False),
    out_type=jax.ShapeDtypeStruct((2 * NVOX, C), jnp.float32),
    scratch_types=[
        pltpu.VMEM_SHARED((TROWS, C), jnp.float32),
        pltpu.VMEM((8, C), jnp.float32),
        pltpu.VMEM((NR,), jnp.int32),
        pltpu.VMEM((NR,), jnp.int32),
        pltpu.VMEM((NR, C), jnp.float32),
        pltpu.SemaphoreType.DMA,
    ],
)


@jax.jit
def kernel(feature2d, depth_mapping_3d):
    feat = feature2d.transpose(0, 2, 3, 1).reshape(B * HW, C)
    out = _sc_call(feat, depth_mapping_3d.reshape(B * NVOX))
    out = out.reshape(NVOX, B, C).transpose(1, 2, 0)
    return out.reshape(B, C, 60, 36, 60)


# speedup vs baseline: 4.6078x; 1.3709x over previous
import functools

import jax
import jax.numpy as jnp
from jax import lax
from jax.experimental import pallas as pl
from jax.experimental.pallas import tpu as pltpu
from jax.experimental.pallas import tpu_sc as plsc

B, C, H, W = 2, 128, 60, 80
HW = H * W
NVOX = 60 * 36 * 60
NTILES = 32
ROWS1 = 4808
TROWS = 2 * ROWS1
VCH = 96
NR = 2 * VCH
NCHUNK = NVOX // VCH
NITER = 44
GSPLITS = (0, 128)


def _sc_body(feat, idx, out, table_s, zbuf, idxb, cidx, rows_v,
             gsem, isem, osem0, osem1):
    cid = lax.axis_index("c")
    sid = lax.axis_index("s")
    wid = sid * 2 + cid

    zeros16 = jnp.zeros((16,), jnp.float32)
    for i in range(8):
        for j in range(8):
            zbuf[i, pl.ds(j * 16, 16)] = zeros16
    bsel = jnp.where(sid < 8, 0, 1)
    src0 = pl.multiple_of(bsel * HW + (sid % 8) * 600, 8)
    dst0 = pl.multiple_of(bsel * ROWS1 + (sid % 8) * 600, 8)
    pltpu.sync_copy(feat.at[pl.ds(src0, 600), :],
                    table_s.at[pl.ds(dst0, 600), :])
    @pl.when(sid == 0)
    def _():
        pltpu.sync_copy(zbuf, table_s.at[pl.ds(HW, 8), :])
        pltpu.sync_copy(zbuf, table_s.at[pl.ds(ROWS1 + HW, 8), :])
    plsc.subcore_barrier()

    iota2 = lax.iota(jnp.int32, 16) * 2
    osem = (osem0, osem1)

    @pl.loop(0, NITER, step=2)
    def round_pair(t0):
        for p in range(2):
            t = t0 + p
            ck = wid + NTILES * t

            @pl.when(ck < NCHUNK)
            def _(p=p, t=t, ck=ck):
                v0 = ck * VCH
                rbase = p * NR
                di0 = pltpu.async_copy(idx.at[pl.ds(v0, VCH)],
                                       idxb.at[pl.ds(rbase, VCH)], isem)
                di1 = pltpu.async_copy(idx.at[pl.ds(NVOX + v0, VCH)],
                                       idxb.at[pl.ds(rbase + VCH, VCH)],
                                       isem)

                @pl.when(t >= 2)
                def _():
                    pltpu.make_async_copy(rows_v.at[pl.ds(rbase, NR), :],
                                          out.at[pl.ds(0, NR), :],
                                          osem[p]).wait()
                di0.wait()
                di1.wait()

                def build_body(g, carry2):
                    pos = iota2 + g * 32
                    iv0 = idxb[pl.ds(rbase + g * 16, 16)]
                    iv1 = idxb[pl.ds(rbase + VCH + g * 16, 16)] + ROWS1
                    plsc.store_scatter(cidx, [pos], iv0)
                    plsc.store_scatter(cidx, [pos + 1], iv1)
                    return carry2

                lax.fori_loop(0, VCH // 16, build_body, 0)

                descs = [pltpu.async_copy(
                    table_s.at[cidx.at[pl.ds(g0, min(128, NR - g0))]],
                    rows_v.at[pl.ds(rbase + g0, min(128, NR - g0)), :],
                    gsem) for g0 in GSPLITS]
                for d in descs:
                    d.wait()
                pltpu.async_copy(rows_v.at[pl.ds(rbase, NR), :],
                                 out.at[pl.ds(v0 * 2, NR), :], osem[p])

    for p in range(2):
        pltpu.make_async_copy(rows_v.at[pl.ds(p * NR, NR), :],
                              out.at[pl.ds(0, NR), :], osem[p]).wait()


_sc_call = pl.kernel(
    _sc_body,
    mesh=plsc.VectorSubcoreMesh(core_axis_name="c", subcore_axis_name="s"),
    compiler_params=pltpu.CompilerParams(needs_layout_passes=False),
    out_type=jax.ShapeDtypeStruct((2 * NVOX, C), jnp.float32),
    scratch_types=[
        pltpu.VMEM_SHARED((TROWS, C), jnp.float32),
        pltpu.VMEM((8, C), jnp.float32),
        pltpu.VMEM((2 * NR,), jnp.int32),
        pltpu.VMEM((NR,), jnp.int32),
        pltpu.VMEM((2 * NR, C), jnp.float32),
        pltpu.SemaphoreType.DMA,
        pltpu.SemaphoreType.DMA,
        pltpu.SemaphoreType.DMA,
        pltpu.SemaphoreType.DMA,
    ],
)


@jax.jit
def kernel(feature2d, depth_mapping_3d):
    feat = feature2d.transpose(0, 2, 3, 1).reshape(B * HW, C)
    out = _sc_call(feat, depth_mapping_3d.reshape(B * NVOX))
    out = out.reshape(NVOX, B, C).transpose(1, 2, 0)
    return out.reshape(B, C, 60, 36, 60)


# speedup vs baseline: 5.5539x; 1.2053x over previous
import functools

import jax
import jax.numpy as jnp
from jax import lax
from jax.experimental import pallas as pl
from jax.experimental.pallas import tpu as pltpu
from jax.experimental.pallas import tpu_sc as plsc

B, C, H, W = 2, 128, 60, 80
HW = H * W
NVOX = 60 * 36 * 60
NTILES = 32
ROWS1 = 4808
TROWS = 2 * ROWS1
VCH = 96
NR = 2 * VCH
NCHUNK = NVOX // VCH
NITER = 44
GSPLITS = (0, 128)


def _sc_body(feat, idx, out, table_s, zbuf, idxb, cidx, rows_v,
             gsem, isem0, isem1, osem0, osem1):
    cid = lax.axis_index("c")
    sid = lax.axis_index("s")
    wid = sid * 2 + cid

    zeros16 = jnp.zeros((16,), jnp.float32)
    for i in range(8):
        for j in range(8):
            zbuf[i, pl.ds(j * 16, 16)] = zeros16
    bsel = jnp.where(sid < 8, 0, 1)
    src0 = pl.multiple_of(bsel * HW + (sid % 8) * 600, 8)
    dst0 = pl.multiple_of(bsel * ROWS1 + (sid % 8) * 600, 8)
    pltpu.sync_copy(feat.at[pl.ds(src0, 600), :],
                    table_s.at[pl.ds(dst0, 600), :])
    @pl.when(sid == 0)
    def _():
        pltpu.sync_copy(zbuf, table_s.at[pl.ds(HW, 8), :])
        pltpu.sync_copy(zbuf, table_s.at[pl.ds(ROWS1 + HW, 8), :])
    plsc.subcore_barrier()

    iota2 = lax.iota(jnp.int32, 16) * 2
    osem = (osem0, osem1)
    isem = (isem0, isem1)

    def fire_idx(ck, slot):
        v0 = ck * VCH
        rbase = slot * NR
        pltpu.async_copy(idx.at[pl.ds(v0, VCH)],
                         idxb.at[pl.ds(rbase, VCH)], isem[slot])
        pltpu.async_copy(idx.at[pl.ds(NVOX + v0, VCH)],
                         idxb.at[pl.ds(rbase + VCH, VCH)], isem[slot])

    def wait_idx(slot):
        rbase = slot * NR
        pltpu.make_async_copy(idx.at[pl.ds(0, VCH)],
                              idxb.at[pl.ds(rbase, VCH)], isem[slot]).wait()
        pltpu.make_async_copy(idx.at[pl.ds(0, VCH)],
                              idxb.at[pl.ds(rbase + VCH, VCH)],
                              isem[slot]).wait()

    fire_idx(wid, 0)

    @pl.loop(0, NITER, step=2)
    def round_pair(t0):
        for p in range(2):
            t = t0 + p
            ck = wid + NTILES * t

            @pl.when(ck < NCHUNK)
            def _(p=p, t=t, ck=ck):
                v0 = ck * VCH
                rbase = p * NR

                @pl.when(ck + NTILES < NCHUNK)
                def _():
                    fire_idx(ck + NTILES, 1 - p)

                @pl.when(t >= 2)
                def _():
                    pltpu.make_async_copy(rows_v.at[pl.ds(rbase, NR), :],
                                          out.at[pl.ds(0, NR), :],
                                          osem[p]).wait()
                wait_idx(p)

                def build_body(g, carry2):
                    pos = iota2 + g * 32
                    iv0 = idxb[pl.ds(rbase + g * 16, 16)]
                    iv1 = idxb[pl.ds(rbase + VCH + g * 16, 16)] + ROWS1
                    plsc.store_scatter(cidx, [pos], iv0)
                    plsc.store_scatter(cidx, [pos + 1], iv1)
                    return carry2

                lax.fori_loop(0, VCH // 16, build_body, 0)

                descs = [pltpu.async_copy(
                    table_s.at[cidx.at[pl.ds(g0, min(128, NR - g0))]],
                    rows_v.at[pl.ds(rbase + g0, min(128, NR - g0)), :],
                    gsem) for g0 in GSPLITS]
                for d in descs:
                    d.wait()
                pltpu.async_copy(rows_v.at[pl.ds(rbase, NR), :],
                                 out.at[pl.ds(v0 * 2, NR), :], osem[p])

    for p in range(2):
        pltpu.make_async_copy(rows_v.at[pl.ds(p * NR, NR), :],
                              out.at[pl.ds(0, NR), :], osem[p]).wait()


_sc_call = pl.kernel(
    _sc_body,
    mesh=plsc.VectorSubcoreMesh(core_axis_name="c", subcore_axis_name="s"),
    compiler_params=pltpu.CompilerParams(needs_layout_passes=False),
    out_type=jax.ShapeDtypeStruct((2 * NVOX, C), jnp.float32),
    scratch_types=[
        pltpu.VMEM_SHARED((TROWS, C), jnp.float32),
        pltpu.VMEM((8, C), jnp.float32),
        pltpu.VMEM((2 * NR,), jnp.int32),
        pltpu.VMEM((NR,), jnp.int32),
        pltpu.VMEM((2 * NR, C), jnp.float32),
        pltpu.SemaphoreType.DMA,
        pltpu.SemaphoreType.DMA,
        pltpu.SemaphoreType.DMA,
        pltpu.SemaphoreType.DMA,
        pltpu.SemaphoreType.DMA,
    ],
)


@jax.jit
def kernel(feature2d, depth_mapping_3d):
    feat = feature2d.transpose(0, 2, 3, 1).reshape(B * HW, C)
    out = _sc_call(feat, depth_mapping_3d.reshape(B * NVOX))
    out = out.reshape(NVOX, B, C).transpose(1, 2, 0)
    return out.reshape(B, C, 60, 36, 60)
